# baseline (speedup 1.0000x reference)
import functools
import math

import jax
import jax.numpy as jnp
from jax import lax
from jax.experimental import pallas as pl
from jax.experimental.pallas import tpu as pltpu
from jax.experimental.pallas import tpu_sc as plsc

_LANES = 16


def _emb_body(nw, b_per_w, chunk, nchunks, d,
              table_hbm, idx_hbm, out_hbm,
              idx_v, gbuf, sbuf, gsem, ssem):
    scale = jnp.float32(math.sqrt(d))
    wid = lax.axis_index("s") * 2 + lax.axis_index("c")
    base = wid * b_per_w
    pltpu.sync_copy(idx_hbm.at[pl.ds(base, b_per_w)], idx_v)

    def chunk_body(g, carry):
        off = pl.multiple_of(g * chunk, chunk)
        pltpu.async_copy(
            table_hbm.at[idx_v.at[pl.ds(off, chunk)]], gbuf, gsem
        ).wait()

        def row_body(i, c):
            for j in range(d // _LANES):
                sl = pl.ds(j * _LANES, _LANES)
                sbuf[i, sl] = gbuf[i, sl] * scale
            return c

        lax.fori_loop(0, chunk, row_body, 0, unroll=4)
        pltpu.async_copy(
            sbuf, out_hbm.at[pl.ds(base + off, chunk)], ssem
        ).wait()
        return carry

    lax.fori_loop(0, nchunks, chunk_body, 0)


def kernel(tokens, table):
    v, d = table.shape
    idx = tokens.reshape(-1).astype(jnp.int32)
    b = idx.shape[0]
    nw = 32
    b_per_w = b // nw
    chunk = 128
    nchunks = b_per_w // chunk

    mesh = plsc.VectorSubcoreMesh(core_axis_name="c", subcore_axis_name="s")
    f = pl.kernel(
        functools.partial(_emb_body, nw, b_per_w, chunk, nchunks, d),
        mesh=mesh,
        compiler_params=pltpu.CompilerParams(use_tc_tiling_on_sc=False),
        out_type=jax.ShapeDtypeStruct((b, d), jnp.float32),
        scratch_types=[
            pltpu.VMEM((b_per_w,), jnp.int32),
            pltpu.VMEM((chunk, d), jnp.float32),
            pltpu.VMEM((chunk, d), jnp.float32),
            pltpu.SemaphoreType.DMA,
            pltpu.SemaphoreType.DMA,
        ],
    )
    out = f(table, idx)
    return out.reshape(*tokens.shape, d)


# speedup vs baseline: 1.1809x; 1.1809x over previous
import functools
import math

import jax
import jax.numpy as jnp
from jax import lax
from jax.experimental import pallas as pl
from jax.experimental.pallas import tpu as pltpu
from jax.experimental.pallas import tpu_sc as plsc

_LANES = 16


def _emb_body(b_per_w, chunk, nchunks, d,
              table_hbm, idx_hbm, out_hbm,
              idx_v, gbufs, sbufs, gsems, ssems):
    scale = jnp.float32(math.sqrt(d))
    wid = lax.axis_index("s") * 2 + lax.axis_index("c")
    base = wid * b_per_w
    pltpu.sync_copy(idx_hbm.at[pl.ds(base, b_per_w)], idx_v)

    def start_gather(g, b):
        off = pl.multiple_of(g * chunk, chunk)
        pltpu.async_copy(
            table_hbm.at[idx_v.at[pl.ds(off, chunk)]], gbufs[b], gsems[b]
        )

    def wait_gather(b):
        pltpu.make_async_copy(
            table_hbm.at[idx_v.at[pl.ds(0, chunk)]], gbufs[b], gsems[b]
        ).wait()

    def start_write(g, b):
        off = pl.multiple_of(g * chunk, chunk)
        pltpu.async_copy(
            sbufs[b], out_hbm.at[pl.ds(base + off, chunk)], ssems[b]
        )

    def wait_write(b):
        pltpu.make_async_copy(
            sbufs[b], out_hbm.at[pl.ds(base, chunk)], ssems[b]
        ).wait()

    def do_scale(b):
        gbuf, sbuf = gbufs[b], sbufs[b]

        def row_body(i, c):
            for j in range(d // _LANES):
                sl = pl.ds(j * _LANES, _LANES)
                sbuf[i, sl] = gbuf[i, sl] * scale
            return c

        lax.fori_loop(0, chunk, row_body, 0, unroll=4)

    start_gather(0, 0)
    start_gather(1, 1)

    for b in range(2):
        wait_gather(b)
        do_scale(b)
        start_gather(2 + b, b)
        start_write(b, b)

    def round_body(r, carry):
        g0 = r * 2
        for b in range(2):
            g = g0 + b
            wait_gather(b)
            wait_write(b)
            do_scale(b)
            start_gather(g + 2, b)
            start_write(g, b)
        return carry

    lax.fori_loop(1, nchunks // 2 - 1, round_body, 0)

    for b in range(2):
        g = nchunks - 2 + b
        wait_gather(b)
        wait_write(b)
        do_scale(b)
        start_write(g, b)
    for b in range(2):
        wait_write(b)


def kernel(tokens, table):
    v, d = table.shape
    idx = tokens.reshape(-1).astype(jnp.int32)
    b = idx.shape[0]
    nw = 32
    b_per_w = b // nw
    chunk = 128
    nchunks = b_per_w // chunk

    mesh = plsc.VectorSubcoreMesh(core_axis_name="c", subcore_axis_name="s")
    f = pl.kernel(
        functools.partial(_emb_body, b_per_w, chunk, nchunks, d),
        mesh=mesh,
        compiler_params=pltpu.CompilerParams(use_tc_tiling_on_sc=False),
        out_type=jax.ShapeDtypeStruct((b, d), jnp.float32),
        scratch_types=[
            pltpu.VMEM((b_per_w,), jnp.int32),
            [pltpu.VMEM((chunk, d), jnp.float32) for _ in range(2)],
            [pltpu.VMEM((chunk, d), jnp.float32) for _ in range(2)],
            [pltpu.SemaphoreType.DMA for _ in range(2)],
            [pltpu.SemaphoreType.DMA for _ in range(2)],
        ],
    )
    out = f(table, idx)
    return out.reshape(*tokens.shape, d)
